# baseline (speedup 1.0000x reference)
import jax
import jax.numpy as jnp
from jax.experimental import pallas as pl
from jax.experimental.pallas import tpu as pltpu

_LANES = 1024
_BLOCK_ROWS = 1024


def _reduce_body(x_ref, out_ref, acc_o, acc_u):
    i = pl.program_id(0)

    @pl.when(i == 0)
    def _init():
        acc_o[...] = jnp.zeros_like(acc_o)
        acc_u[...] = jnp.zeros_like(acc_u)

    x = x_ref[...]
    t = x - 1.0
    o = jnp.maximum(t, 0.0)
    u = jnp.minimum(x, 0.0)
    o2 = (o * o).reshape(_BLOCK_ROWS // 8, 8, _LANES).sum(axis=0)
    u2 = (u * u).reshape(_BLOCK_ROWS // 8, 8, _LANES).sum(axis=0)
    acc_o[...] += o2
    acc_u[...] += u2

    @pl.when(i == pl.num_programs(0) - 1)
    def _fini():
        s_o = jnp.sum(acc_o[...])
        s_u = jnp.sum(acc_u[...])
        out_ref[0, 0] = jnp.sqrt(s_o) + jnp.sqrt(s_u)


def kernel(tensor):
    n = tensor.size
    rows = n // _LANES
    x2d = tensor.reshape(rows, _LANES)
    grid = rows // _BLOCK_ROWS

    out = pl.pallas_call(
        _reduce_body,
        grid=(grid,),
        in_specs=[pl.BlockSpec((_BLOCK_ROWS, _LANES), lambda i: (i, 0))],
        out_specs=pl.BlockSpec(
            (1, 1), lambda i: (0, 0), memory_space=pltpu.SMEM
        ),
        out_shape=jax.ShapeDtypeStruct((1, 1), jnp.float32),
        scratch_shapes=[
            pltpu.VMEM((8, _LANES), jnp.float32),
            pltpu.VMEM((8, _LANES), jnp.float32),
        ],
        compiler_params=pltpu.CompilerParams(
            dimension_semantics=("arbitrary",),
        ),
    )(x2d)
    return out[0, 0]


# speedup vs baseline: 1.0077x; 1.0077x over previous
import jax
import jax.numpy as jnp
from jax.experimental import pallas as pl
from jax.experimental.pallas import tpu as pltpu

_LANES = 1024
_BLOCK_ROWS = 1024


def _reduce_body(x_ref, out_ref, acc_o, acc_u):
    i = pl.program_id(0)

    @pl.when(i == 0)
    def _init():
        acc_o[...] = jnp.zeros_like(acc_o)
        acc_u[...] = jnp.zeros_like(acc_u)

    x = x_ref[...]
    t = x - 1.0
    o = jnp.maximum(t, 0.0)
    u = jnp.minimum(x, 0.0)
    acc_o[...] += o * o
    acc_u[...] += u * u

    @pl.when(i == pl.num_programs(0) - 1)
    def _fini():
        s_o = jnp.sum(acc_o[...])
        s_u = jnp.sum(acc_u[...])
        out_ref[0, 0] = jnp.sqrt(s_o) + jnp.sqrt(s_u)


def kernel(tensor):
    n = tensor.size
    rows = n // _LANES
    x2d = tensor.reshape(rows, _LANES)
    grid = rows // _BLOCK_ROWS

    out = pl.pallas_call(
        _reduce_body,
        grid=(grid,),
        in_specs=[pl.BlockSpec((_BLOCK_ROWS, _LANES), lambda i: (i, 0))],
        out_specs=pl.BlockSpec(
            (1, 1), lambda i: (0, 0), memory_space=pltpu.SMEM
        ),
        out_shape=jax.ShapeDtypeStruct((1, 1), jnp.float32),
        scratch_shapes=[
            pltpu.VMEM((_BLOCK_ROWS, _LANES), jnp.float32),
            pltpu.VMEM((_BLOCK_ROWS, _LANES), jnp.float32),
        ],
        compiler_params=pltpu.CompilerParams(
            dimension_semantics=("arbitrary",),
        ),
    )(x2d)
    return out[0, 0]
